# baseline (speedup 1.0000x reference)
import functools

import jax
import jax.numpy as jnp
from jax import lax
from jax.experimental import pallas as pl
from jax.experimental.pallas import tpu as pltpu
from jax.experimental.pallas import tpu_sc as plsc

HID = 512
NF = 256
EF = 16
N = 10000
E = 160000

_BNS = (1.0 + 1e-5) ** -0.5

SLAB = 128
NSLAB = HID // SLAB
CHUNK = 80
NCHUNK = E // CHUNK
CPT = NCHUNK // 16
RPT = 640
NPAD = 16 * RPT
NL = 4
NGRP = 31

C3 = 64
NC3 = E // C3
NB3 = 3
NGRP3 = 26


def _sc_edge_scatter_body(e0, e1, e2, e3, row_h, col_h, z2_h, z1_h, ones_h,
                          agg_h, cnt_h,
                          evs, ridxs, cidxs, onesv, slab_s, cnt_s,
                          sem_i, sem_e, sem_s):
    cid = lax.axis_index("c")
    sid = lax.axis_index("s")
    r0 = sid * RPT
    c0 = sid * CPT
    e_all = (e0, e1, e2, e3)
    pltpu.sync_copy(ones_h, onesv)

    for p in range(NSLAB):
        @pl.when(cid == p // 2)
        def _pass(p=p):
            e_p = e_all[p]
            pltpu.sync_copy(z2_h, slab_s.at[pl.ds(r0, RPT), :])
            if p == 0:
                pltpu.sync_copy(z1_h, cnt_s.at[pl.ds(r0, RPT)])
            plsc.subcore_barrier()

            def _group(g, carry):
                fires = []
                for b in range(NL):
                    @pl.when(g > 0)
                    def _drain(b=b):
                        pltpu.make_async_copy(
                            evs[b], slab_s.at[ridxs[b]], sem_s).wait()
                        pltpu.make_async_copy(
                            evs[b], slab_s.at[cidxs[b]], sem_s).wait()
                    base = (c0 + g * NL + b) * CHUNK
                    fires.append(pltpu.async_copy(
                        row_h.at[pl.ds(base, CHUNK)], ridxs[b], sem_i))
                    fires.append(pltpu.async_copy(
                        col_h.at[pl.ds(base, CHUNK)], cidxs[b], sem_i))
                    fires.append(pltpu.async_copy(
                        e_p.at[pl.ds(base, CHUNK)], evs[b], sem_e))
                for f in fires:
                    f.wait()
                for b in range(NL):
                    pltpu.async_copy(
                        evs[b], slab_s.at[ridxs[b]], sem_s, add=True)
                    pltpu.async_copy(
                        evs[b], slab_s.at[cidxs[b]], sem_s, add=True)
                return carry

            lax.fori_loop(0, NGRP, _group, 0)
            for b in range(NL):
                pltpu.make_async_copy(
                    evs[b], slab_s.at[ridxs[b]], sem_s).wait()
                pltpu.make_async_copy(
                    evs[b], slab_s.at[cidxs[b]], sem_s).wait()

            base = (c0 + NL * NGRP) * CHUNK
            pltpu.sync_copy(row_h.at[pl.ds(base, CHUNK)], ridxs[0])
            pltpu.sync_copy(col_h.at[pl.ds(base, CHUNK)], cidxs[0])
            pltpu.sync_copy(e_p.at[pl.ds(base, CHUNK)], evs[0])
            pltpu.sync_copy(evs[0], slab_s.at[ridxs[0]], add=True)
            pltpu.sync_copy(evs[0], slab_s.at[cidxs[0]], add=True)

            plsc.subcore_barrier()

            @pl.when(sid < 15)
            def _write_full():
                pltpu.sync_copy(
                    slab_s.at[pl.ds(r0, RPT), :],
                    agg_h.at[pl.ds(r0, RPT), pl.ds(p * SLAB, SLAB)])

            @pl.when(sid == 15)
            def _write_tail():
                nr = N - 15 * RPT
                pltpu.sync_copy(
                    slab_s.at[pl.ds(15 * RPT, nr), :],
                    agg_h.at[pl.ds(15 * RPT, nr), pl.ds(p * SLAB, SLAB)])

            plsc.subcore_barrier()

    @pl.when(cid == 0)
    def _cnt():
        def _group(g, carry):
            fires = []
            for b in range(NL):
                base = (c0 + g * NL + b) * CHUNK
                fires.append(pltpu.async_copy(
                    row_h.at[pl.ds(base, CHUNK)], ridxs[b], sem_i))
                fires.append(pltpu.async_copy(
                    col_h.at[pl.ds(base, CHUNK)], cidxs[b], sem_i))
            for f in fires:
                f.wait()
            scat = []
            for b in range(NL):
                scat.append(pltpu.async_copy(
                    onesv, cnt_s.at[ridxs[b]], sem_s, add=True))
                scat.append(pltpu.async_copy(
                    onesv, cnt_s.at[cidxs[b]], sem_s, add=True))
            for f in scat:
                f.wait()
            return carry

        lax.fori_loop(0, NGRP, _group, 0)
        base = (c0 + NL * NGRP) * CHUNK
        pltpu.sync_copy(row_h.at[pl.ds(base, CHUNK)], ridxs[0])
        pltpu.sync_copy(col_h.at[pl.ds(base, CHUNK)], cidxs[0])
        pltpu.sync_copy(onesv, cnt_s.at[ridxs[0]], add=True)
        pltpu.sync_copy(onesv, cnt_s.at[cidxs[0]], add=True)
        plsc.subcore_barrier()
        pltpu.sync_copy(cnt_s.at[pl.ds(r0, RPT)], cnt_h.at[pl.ds(r0, RPT)])


def _sc_gin_agg_body(h4_h, row_h, col_h, z2_h,
                     nbr_h,
                     evs, ridxs, cidxs, gidxs, slab_s,
                     sem_i, sem_g, sem_s):
    cid = lax.axis_index("c")
    sid = lax.axis_index("s")
    r0 = sid * RPT
    c0 = sid * CPT

    for p in range(NSLAB):
        @pl.when(cid == p // 2)
        def _pass(p=p):
            pltpu.sync_copy(z2_h, slab_s.at[pl.ds(r0, RPT), :])
            plsc.subcore_barrier()

            def _group(g, carry):
                ifires = []
                for b in range(NL):
                    @pl.when(g > 0)
                    def _drain(b=b):
                        pltpu.make_async_copy(
                            evs[b], slab_s.at[cidxs[b]], sem_s).wait()
                    base = (c0 + g * NL + b) * CHUNK
                    ifires.append(pltpu.async_copy(
                        row_h.at[pl.ds(base, CHUNK)], ridxs[b], sem_i))
                    ifires.append(pltpu.async_copy(
                        col_h.at[pl.ds(base, CHUNK)], cidxs[b], sem_i))
                for f in ifires:
                    f.wait()
                gath = []
                for b in range(NL):
                    for k in range(CHUNK // 16):
                        gidxs[b][pl.ds(k * 16, 16)] = (
                            ridxs[b][pl.ds(k * 16, 16)] * 4 + p)
                    gath.append(pltpu.async_copy(
                        h4_h.at[gidxs[b]], evs[b], sem_g))
                for b in range(NL):
                    gath[b].wait()
                    pltpu.async_copy(
                        evs[b], slab_s.at[cidxs[b]], sem_s, add=True)
                return carry

            lax.fori_loop(0, NGRP, _group, 0)
            for b in range(NL):
                pltpu.make_async_copy(
                    evs[b], slab_s.at[cidxs[b]], sem_s).wait()

            base = (c0 + NL * NGRP) * CHUNK
            pltpu.sync_copy(row_h.at[pl.ds(base, CHUNK)], ridxs[0])
            pltpu.sync_copy(col_h.at[pl.ds(base, CHUNK)], cidxs[0])
            for k in range(CHUNK // 16):
                gidxs[0][pl.ds(k * 16, 16)] = (
                    ridxs[0][pl.ds(k * 16, 16)] * 4 + p)
            pltpu.async_copy(h4_h.at[gidxs[0]], evs[0], sem_g).wait()
            pltpu.sync_copy(evs[0], slab_s.at[cidxs[0]], add=True)

            plsc.subcore_barrier()

            @pl.when(sid < 15)
            def _write_full():
                pltpu.sync_copy(
                    slab_s.at[pl.ds(r0, RPT), :],
                    nbr_h.at[pl.ds(r0, RPT), pl.ds(p * SLAB, SLAB)])

            @pl.when(sid == 15)
            def _write_tail():
                nr = N - 15 * RPT
                pltpu.sync_copy(
                    slab_s.at[pl.ds(15 * RPT, nr), :],
                    nbr_h.at[pl.ds(15 * RPT, nr), pl.ds(p * SLAB, SLAB)])

            plsc.subcore_barrier()


def _sc_edge_gather_body(hs_h, hd_h, row_h, col_h,
                         s_out_h, d_out_h,
                         bufs, idxs,
                         sem_i, sem_g, sem_w):
    cid = lax.axis_index("c")
    sid = lax.axis_index("s")
    wid = sid * 2 + cid

    for table, idx_h, out_h in ((hs_h, row_h, s_out_h),
                                (hd_h, col_h, d_out_h)):
        def _group(g, carry, table=table, idx_h=idx_h, out_h=out_h):
            fires = []
            for b in range(NB3):
                base = (wid + (g * NB3 + b) * 32) * C3
                fires.append(pltpu.async_copy(
                    idx_h.at[pl.ds(base, C3)], idxs[b], sem_i))
            for f in fires:
                f.wait()
            gath = [pltpu.async_copy(table.at[idxs[b]], bufs[b], sem_g)
                    for b in range(NB3)]
            for f in gath:
                f.wait()
            wr = []
            for b in range(NB3):
                base = (wid + (g * NB3 + b) * 32) * C3
                wr.append(pltpu.async_copy(
                    bufs[b], out_h.at[pl.ds(base, C3)], sem_w))
            for f in wr:
                f.wait()
            return carry

        lax.fori_loop(0, NGRP3, _group, 0)

        @pl.when(wid <= 3)
        def _tail(table=table, idx_h=idx_h, out_h=out_h):
            base = (2496 + wid) * C3
            pltpu.sync_copy(idx_h.at[pl.ds(base, C3)], idxs[0])
            pltpu.async_copy(table.at[idxs[0]], bufs[0], sem_g).wait()
            pltpu.sync_copy(bufs[0], out_h.at[pl.ds(base, C3)])


@functools.lru_cache(maxsize=1)
def _sc_kernels():
    mesh = plsc.VectorSubcoreMesh(core_axis_name="c", subcore_axis_name="s")

    def vmems(n, shape, dtype):
        return [pltpu.VMEM(shape, dtype) for _ in range(n)]

    def edge_scatter_wrap(es, row, col, z2, z1, ones1):
        body = lambda *args: _sc_edge_scatter_body(
            *args[:11],
            list(args[11:11 + NL]), list(args[11 + NL:11 + 2 * NL]),
            list(args[11 + 2 * NL:11 + 3 * NL]),
            *args[11 + 3 * NL:])
        return pl.kernel(
            body,
            mesh=mesh,
            out_type=(
                jax.ShapeDtypeStruct((N, HID), jnp.float32),
                jax.ShapeDtypeStruct((NPAD,), jnp.float32),
            ),
            scratch_types=(
                vmems(NL, (CHUNK, SLAB), jnp.float32)
                + vmems(NL, (CHUNK,), jnp.int32)
                + vmems(NL, (CHUNK,), jnp.int32)
                + [pltpu.VMEM((CHUNK,), jnp.float32),
                   pltpu.VMEM_SHARED((NPAD, SLAB), jnp.float32),
                   pltpu.VMEM_SHARED((NPAD,), jnp.float32),
                   pltpu.SemaphoreType.DMA, pltpu.SemaphoreType.DMA,
                   pltpu.SemaphoreType.DMA]
            ),
        )(*es, row, col, z2, z1, ones1)

    def gin_agg_wrap(h4, row, col, z2):
        body = lambda *args: _sc_gin_agg_body(
            *args[:5],
            list(args[5:5 + NL]), list(args[5 + NL:5 + 2 * NL]),
            list(args[5 + 2 * NL:5 + 3 * NL]),
            list(args[5 + 3 * NL:5 + 4 * NL]),
            *args[5 + 4 * NL:])
        return pl.kernel(
            body,
            mesh=mesh,
            out_type=jax.ShapeDtypeStruct((N, HID), jnp.float32),
            scratch_types=(
                vmems(NL, (CHUNK, SLAB), jnp.float32)
                + vmems(NL, (CHUNK,), jnp.int32)
                + vmems(NL, (CHUNK,), jnp.int32)
                + vmems(NL, (CHUNK,), jnp.int32)
                + [pltpu.VMEM_SHARED((NPAD, SLAB), jnp.float32),
                   pltpu.SemaphoreType.DMA, pltpu.SemaphoreType.DMA,
                   pltpu.SemaphoreType.DMA]
            ),
        )(h4, row, col, z2)

    def edge_gather_wrap(hs, hd, row, col):
        body = lambda *args: _sc_edge_gather_body(
            *args[:6],
            list(args[6:6 + NB3]), list(args[6 + NB3:6 + 2 * NB3]),
            *args[6 + 2 * NB3:])
        return pl.kernel(
            body,
            mesh=mesh,
            out_type=(
                jax.ShapeDtypeStruct((E, HID), jnp.float32),
                jax.ShapeDtypeStruct((E, HID), jnp.float32),
            ),
            scratch_types=(
                vmems(NB3, (C3, HID), jnp.float32)
                + vmems(NB3, (C3,), jnp.int32)
                + [pltpu.SemaphoreType.DMA, pltpu.SemaphoreType.DMA,
                   pltpu.SemaphoreType.DMA]
            ),
        )(hs, hd, row, col)

    return edge_scatter_wrap, gin_agg_wrap, edge_gather_wrap


def _dot(a, b):
    return jnp.dot(a, b, preferred_element_type=jnp.float32,
                   precision=lax.Precision.HIGHEST)


def _full(shape):
    return pl.BlockSpec(shape, lambda i: (0, 0))


def _mlp2_body(x_ref, w1_ref, b1_ref, w2_ref, b2_ref, o_ref):
    z = jnp.maximum(_dot(x_ref[...], w1_ref[...]) + b1_ref[...], 0.0)
    o_ref[...] = _dot(z, w2_ref[...]) + b2_ref[...]


def _mlp2(x, w1, b1, w2, b2, blk):
    m, k = x.shape
    return pl.pallas_call(
        _mlp2_body,
        grid=(m // blk,),
        in_specs=[
            pl.BlockSpec((blk, k), lambda i: (i, 0)),
            _full(w1.shape), _full(b1.shape),
            _full(w2.shape), _full(b2.shape),
        ],
        out_specs=pl.BlockSpec((blk, w2.shape[1]), lambda i: (i, 0)),
        out_shape=jax.ShapeDtypeStruct((m, w2.shape[1]), jnp.float32),
    )(x, w1, b1, w2, b2)


def _edge_enc_body(x_ref, w_ref, b_ref, *o_refs):
    z = jnp.maximum(_dot(x_ref[...], w_ref[...]) + b_ref[...], 0.0)
    for k, o_ref in enumerate(o_refs):
        o_ref[...] = z[:, k * SLAB:(k + 1) * SLAB]


def _edge_enc(x, w, b, blk):
    spec = pl.BlockSpec((blk, SLAB), lambda i: (i, 0))
    shp = jax.ShapeDtypeStruct((E, SLAB), jnp.float32)
    return pl.pallas_call(
        _edge_enc_body,
        grid=(E // blk,),
        in_specs=[
            pl.BlockSpec((blk, EF), lambda i: (i, 0)),
            _full(w.shape), _full(b.shape),
        ],
        out_specs=(spec,) * NSLAB,
        out_shape=(shp,) * NSLAB,
    )(x, w, b)


def _fusion_body(h_ref, agg_ref, cnt_ref, w1a_ref, w1b_ref, b1_ref,
                 w2_ref, b2_ref, o_ref):
    cnt = jnp.maximum(cnt_ref[...], 1.0)
    agg = agg_ref[...] / cnt
    z = jnp.maximum(_dot(h_ref[...], w1a_ref[...])
                    + _dot(agg, w1b_ref[...]) + b1_ref[...], 0.0)
    o_ref[...] = _dot(z, w2_ref[...]) + b2_ref[...]


def _fusion(h, agg, cnt, w1a, w1b, b1, w2, b2, blk):
    return pl.pallas_call(
        _fusion_body,
        grid=(N // blk,),
        in_specs=[
            pl.BlockSpec((blk, HID), lambda i: (i, 0)),
            pl.BlockSpec((blk, HID), lambda i: (i, 0)),
            pl.BlockSpec((blk, 1), lambda i: (i, 0)),
            _full(w1a.shape), _full(w1b.shape), _full(b1.shape),
            _full(w2.shape), _full(b2.shape),
        ],
        out_specs=pl.BlockSpec((blk, HID), lambda i: (i, 0)),
        out_shape=jax.ShapeDtypeStruct((N, HID), jnp.float32),
    )(h, agg, cnt, w1a, w1b, b1, w2, b2)


def _gin_body(h_ref, nbr_ref, eps_ref, w1_ref, b1_ref, w2_ref, b2_ref,
              a3_ref, c3_ref, o_ref):
    z = eps_ref[...] * h_ref[...] + nbr_ref[...]
    z = jnp.maximum(_dot(z, w1_ref[...]) + b1_ref[...], 0.0)
    z = jnp.maximum(_dot(z, w2_ref[...]) + b2_ref[...], 0.0)
    o_ref[...] = z * a3_ref[...] + c3_ref[...]


def _gin_mlp(h, nbr, epsp, w1, b1, w2, b2, a3, c3, blk):
    return pl.pallas_call(
        _gin_body,
        grid=(N // blk,),
        in_specs=[
            pl.BlockSpec((blk, HID), lambda i: (i, 0)),
            pl.BlockSpec((blk, HID), lambda i: (i, 0)),
            _full(epsp.shape),
            _full(w1.shape), _full(b1.shape),
            _full(w2.shape), _full(b2.shape),
            _full(a3.shape), _full(c3.shape),
        ],
        out_specs=pl.BlockSpec((blk, HID), lambda i: (i, 0)),
        out_shape=jax.ShapeDtypeStruct((N, HID), jnp.float32),
    )(h, nbr, epsp, w1, b1, w2, b2, a3, c3)


def _prep_body(h_ref, ws_ref, wd_ref, os_ref, od_ref):
    h = h_ref[...]
    os_ref[...] = _dot(h, ws_ref[...])
    od_ref[...] = _dot(h, wd_ref[...])


def _prep(h, ws, wd, blk):
    return pl.pallas_call(
        _prep_body,
        grid=(N // blk,),
        in_specs=[
            pl.BlockSpec((blk, HID), lambda i: (i, 0)),
            _full(ws.shape), _full(wd.shape),
        ],
        out_specs=(
            pl.BlockSpec((blk, HID), lambda i: (i, 0)),
            pl.BlockSpec((blk, HID), lambda i: (i, 0)),
        ),
        out_shape=(
            jax.ShapeDtypeStruct((N, HID), jnp.float32),
            jax.ShapeDtypeStruct((N, HID), jnp.float32),
        ),
    )(h, ws, wd)


def _cls_body(s_ref, d_ref, b1_ref, w2_ref, b2_ref, w3_ref, b3_ref,
              w4_ref, b4_ref, o_ref):
    z = jnp.maximum(s_ref[...] + d_ref[...] + b1_ref[...], 0.0)
    z = jnp.maximum(_dot(z, w2_ref[...]) + b2_ref[...], 0.0)
    z = jnp.maximum(_dot(z, w3_ref[...]) + b3_ref[...], 0.0)
    o_ref[...] = _dot(z, w4_ref[...]) + b4_ref[...]


def _cls(s, d, b1, w2, b2, w3, b3, w4, b4, blk):
    return pl.pallas_call(
        _cls_body,
        grid=(E // blk,),
        in_specs=[
            pl.BlockSpec((blk, HID), lambda i: (i, 0)),
            pl.BlockSpec((blk, HID), lambda i: (i, 0)),
            _full(b1.shape),
            _full(w2.shape), _full(b2.shape),
            _full(w3.shape), _full(b3.shape),
            _full(w4.shape), _full(b4.shape),
        ],
        out_specs=pl.BlockSpec((blk, 2), lambda i: (i, 0)),
        out_shape=jax.ShapeDtypeStruct((E, 2), jnp.float32),
    )(s, d, b1, w2, b2, w3, b3, w4, b4)


def kernel(x, edge_index, edge_attr, params):
    p = params
    row = edge_index[0]
    col = edge_index[1]
    r2 = lambda v: v.reshape(1, -1)
    sc_edge_scatter, sc_gin_agg, sc_edge_gather = _sc_kernels()

    ne_s = p['ne_g1'] * _BNS
    h = _mlp2(x, p['ne_W1'] * ne_s[None, :],
              r2(p['ne_b1'] * ne_s + p['ne_be1']),
              p['ne_W2'], r2(p['ne_b2']), blk=1000)

    ee_s = p['ee_g'] * _BNS
    es = _edge_enc(edge_attr, p['ee_W'] * ee_s[None, :],
                   r2(p['ee_b'] * ee_s + p['ee_be']), blk=2000)

    z2 = jnp.zeros((RPT, SLAB), jnp.float32)
    z1 = jnp.zeros((RPT,), jnp.float32)
    ones1 = jnp.ones((CHUNK,), jnp.float32)
    agg, cnt_pad = sc_edge_scatter(es, row, col, z2, z1, ones1)
    cnt = cnt_pad[:N].reshape(N, 1)

    h = _fusion(h, agg, cnt,
                p['ef_W1'][:HID], p['ef_W1'][HID:], r2(p['ef_b1']),
                p['ef_W2'], r2(p['ef_b2']), blk=1000)

    for i in range(3):
        nbr = sc_gin_agg(h.reshape(NSLAB * N, SLAB), row, col, z2)
        s1 = p['g%d_g1' % i] * _BNS
        s2 = p['g%d_g2' % i] * _BNS
        h = _gin_mlp(
            h, nbr, (1.0 + p['eps%d' % i]).reshape(1, 1),
            p['g%d_W1' % i] * s1[None, :],
            r2(p['g%d_b1' % i] * s1 + p['g%d_be1' % i]),
            p['g%d_W2' % i] * s2[None, :],
            r2(p['g%d_b2' % i] * s2 + p['g%d_be2' % i]),
            r2(p['bn%d_g' % i] * _BNS), r2(p['bn%d_b' % i]), blk=1000)

    c1 = p['c_g1'] * _BNS
    wt, wb = p['c_W1'][:HID], p['c_W1'][HID:]
    hs, hd = _prep(h, (wt + wb) * c1[None, :], (wt - wb) * c1[None, :],
                   blk=1000)

    s_rows, d_rows = sc_edge_gather(hs, hd, row, col)

    c2 = p['c_g2'] * _BNS
    out = _cls(s_rows, d_rows,
               r2(p['c_b1'] * c1 + p['c_be1']),
               p['c_W2'] * c2[None, :], r2(p['c_b2'] * c2 + p['c_be2']),
               p['c_W3'], r2(p['c_b3']),
               p['c_W4'], r2(p['c_b4']), blk=2000)
    return out


# speedup vs baseline: 1.5098x; 1.5098x over previous
import functools

import jax
import jax.numpy as jnp
from jax import lax
from jax.experimental import pallas as pl
from jax.experimental.pallas import tpu as pltpu
from jax.experimental.pallas import tpu_sc as plsc

HID = 512
NF = 256
EF = 16
N = 10000
E = 160000

_BNS = (1.0 + 1e-5) ** -0.5

SLAB = 128
NSLAB = HID // SLAB
CHUNK = 80
NCHUNK = E // CHUNK
CPT = NCHUNK // 16
RPT = 640
NPAD = 16 * RPT
NL = 4
NGRP = 31

C3 = 64
E2 = E // 2
NC3 = E2 // C3
NB3 = 3
NGRP3 = 13


def _sc_edge_scatter_body(e0, e1, e2, e3, row_h, col_h, z2_h, z1_h, ones_h,
                          agg_h, cnt_h,
                          evs, ridxs, cidxs, onesv, slab_s, cnt_s,
                          sem_i, sem_e, sem_s):
    cid = lax.axis_index("c")
    sid = lax.axis_index("s")
    r0 = sid * RPT
    c0 = sid * CPT
    e_all = (e0, e1, e2, e3)
    pltpu.sync_copy(ones_h, onesv)

    for p in range(NSLAB):
        @pl.when(cid == p // 2)
        def _pass(p=p):
            e_p = e_all[p]
            pltpu.sync_copy(z2_h, slab_s.at[pl.ds(r0, RPT), :])
            if p == 0:
                pltpu.sync_copy(z1_h, cnt_s.at[pl.ds(r0, RPT)])
            plsc.subcore_barrier()

            def _group(g, carry):
                fires = []
                for b in range(NL):
                    @pl.when(g > 0)
                    def _drain(b=b):
                        pltpu.make_async_copy(
                            evs[b], slab_s.at[ridxs[b]], sem_s).wait()
                        pltpu.make_async_copy(
                            evs[b], slab_s.at[cidxs[b]], sem_s).wait()
                    base = (c0 + g * NL + b) * CHUNK
                    fires.append(pltpu.async_copy(
                        row_h.at[pl.ds(base, CHUNK)], ridxs[b], sem_i))
                    fires.append(pltpu.async_copy(
                        col_h.at[pl.ds(base, CHUNK)], cidxs[b], sem_i))
                    fires.append(pltpu.async_copy(
                        e_p.at[pl.ds(base, CHUNK)], evs[b], sem_e))
                for f in fires:
                    f.wait()
                for b in range(NL):
                    pltpu.async_copy(
                        evs[b], slab_s.at[ridxs[b]], sem_s, add=True)
                    pltpu.async_copy(
                        evs[b], slab_s.at[cidxs[b]], sem_s, add=True)
                return carry

            lax.fori_loop(0, NGRP, _group, 0)
            for b in range(NL):
                pltpu.make_async_copy(
                    evs[b], slab_s.at[ridxs[b]], sem_s).wait()
                pltpu.make_async_copy(
                    evs[b], slab_s.at[cidxs[b]], sem_s).wait()

            base = (c0 + NL * NGRP) * CHUNK
            pltpu.sync_copy(row_h.at[pl.ds(base, CHUNK)], ridxs[0])
            pltpu.sync_copy(col_h.at[pl.ds(base, CHUNK)], cidxs[0])
            pltpu.sync_copy(e_p.at[pl.ds(base, CHUNK)], evs[0])
            pltpu.sync_copy(evs[0], slab_s.at[ridxs[0]], add=True)
            pltpu.sync_copy(evs[0], slab_s.at[cidxs[0]], add=True)

            plsc.subcore_barrier()

            @pl.when(sid < 15)
            def _write_full():
                pltpu.sync_copy(
                    slab_s.at[pl.ds(r0, RPT), :],
                    agg_h.at[pl.ds(r0, RPT), pl.ds(p * SLAB, SLAB)])

            @pl.when(sid == 15)
            def _write_tail():
                nr = N - 15 * RPT
                pltpu.sync_copy(
                    slab_s.at[pl.ds(15 * RPT, nr), :],
                    agg_h.at[pl.ds(15 * RPT, nr), pl.ds(p * SLAB, SLAB)])

            plsc.subcore_barrier()

    @pl.when(cid == 0)
    def _cnt():
        def _group(g, carry):
            fires = []
            for b in range(NL):
                base = (c0 + g * NL + b) * CHUNK
                fires.append(pltpu.async_copy(
                    row_h.at[pl.ds(base, CHUNK)], ridxs[b], sem_i))
                fires.append(pltpu.async_copy(
                    col_h.at[pl.ds(base, CHUNK)], cidxs[b], sem_i))
            for f in fires:
                f.wait()
            scat = []
            for b in range(NL):
                scat.append(pltpu.async_copy(
                    onesv, cnt_s.at[ridxs[b]], sem_s, add=True))
                scat.append(pltpu.async_copy(
                    onesv, cnt_s.at[cidxs[b]], sem_s, add=True))
            for f in scat:
                f.wait()
            return carry

        lax.fori_loop(0, NGRP, _group, 0)
        base = (c0 + NL * NGRP) * CHUNK
        pltpu.sync_copy(row_h.at[pl.ds(base, CHUNK)], ridxs[0])
        pltpu.sync_copy(col_h.at[pl.ds(base, CHUNK)], cidxs[0])
        pltpu.sync_copy(onesv, cnt_s.at[ridxs[0]], add=True)
        pltpu.sync_copy(onesv, cnt_s.at[cidxs[0]], add=True)
        plsc.subcore_barrier()
        pltpu.sync_copy(cnt_s.at[pl.ds(r0, RPT)], cnt_h.at[pl.ds(r0, RPT)])


def _sc_gin_agg_body(h4_h, row_h, col_h, z2_h,
                     nbr_h,
                     evs, ridxs, cidxs, gidxs, slab_s,
                     sem_i, sem_g, sem_s):
    cid = lax.axis_index("c")
    sid = lax.axis_index("s")
    r0 = sid * RPT
    c0 = sid * CPT

    for p in range(NSLAB):
        @pl.when(cid == p // 2)
        def _pass(p=p):
            pltpu.sync_copy(z2_h, slab_s.at[pl.ds(r0, RPT), :])
            plsc.subcore_barrier()

            def _group(g, carry):
                ifires = []
                for b in range(NL):
                    @pl.when(g > 0)
                    def _drain(b=b):
                        pltpu.make_async_copy(
                            evs[b], slab_s.at[cidxs[b]], sem_s).wait()
                    base = (c0 + g * NL + b) * CHUNK
                    ifires.append(pltpu.async_copy(
                        row_h.at[pl.ds(base, CHUNK)], ridxs[b], sem_i))
                    ifires.append(pltpu.async_copy(
                        col_h.at[pl.ds(base, CHUNK)], cidxs[b], sem_i))
                for f in ifires:
                    f.wait()
                gath = []
                for b in range(NL):
                    for k in range(CHUNK // 16):
                        gidxs[b][pl.ds(k * 16, 16)] = (
                            ridxs[b][pl.ds(k * 16, 16)] * 4 + p)
                    gath.append(pltpu.async_copy(
                        h4_h.at[gidxs[b]], evs[b], sem_g))
                for b in range(NL):
                    gath[b].wait()
                    pltpu.async_copy(
                        evs[b], slab_s.at[cidxs[b]], sem_s, add=True)
                return carry

            lax.fori_loop(0, NGRP, _group, 0)
            for b in range(NL):
                pltpu.make_async_copy(
                    evs[b], slab_s.at[cidxs[b]], sem_s).wait()

            base = (c0 + NL * NGRP) * CHUNK
            pltpu.sync_copy(row_h.at[pl.ds(base, CHUNK)], ridxs[0])
            pltpu.sync_copy(col_h.at[pl.ds(base, CHUNK)], cidxs[0])
            for k in range(CHUNK // 16):
                gidxs[0][pl.ds(k * 16, 16)] = (
                    ridxs[0][pl.ds(k * 16, 16)] * 4 + p)
            pltpu.async_copy(h4_h.at[gidxs[0]], evs[0], sem_g).wait()
            pltpu.sync_copy(evs[0], slab_s.at[cidxs[0]], add=True)

            plsc.subcore_barrier()

            @pl.when(sid < 15)
            def _write_full():
                pltpu.sync_copy(
                    slab_s.at[pl.ds(r0, RPT), :],
                    nbr_h.at[pl.ds(r0, RPT), pl.ds(p * SLAB, SLAB)])

            @pl.when(sid == 15)
            def _write_tail():
                nr = N - 15 * RPT
                pltpu.sync_copy(
                    slab_s.at[pl.ds(15 * RPT, nr), :],
                    nbr_h.at[pl.ds(15 * RPT, nr), pl.ds(p * SLAB, SLAB)])

            plsc.subcore_barrier()


def _sc_edge_gather_body(hs_h, hd_h, row_h, col_h,
                         s_out_h, d_out_h,
                         bufs, idxs,
                         sem_i, sem_g, sem_w):
    cid = lax.axis_index("c")
    sid = lax.axis_index("s")
    wid = sid * 2 + cid

    for table, idx_h, out_h in ((hs_h, row_h, s_out_h),
                                (hd_h, col_h, d_out_h)):
        def _group(g, carry, table=table, idx_h=idx_h, out_h=out_h):
            fires = []
            for b in range(NB3):
                base = (wid + (g * NB3 + b) * 32) * C3
                fires.append(pltpu.async_copy(
                    idx_h.at[pl.ds(base, C3)], idxs[b], sem_i))
            for f in fires:
                f.wait()
            gath = [pltpu.async_copy(table.at[idxs[b]], bufs[b], sem_g)
                    for b in range(NB3)]
            for f in gath:
                f.wait()
            wr = []
            for b in range(NB3):
                base = (wid + (g * NB3 + b) * 32) * C3
                wr.append(pltpu.async_copy(
                    bufs[b], out_h.at[pl.ds(base, C3)], sem_w))
            for f in wr:
                f.wait()
            return carry

        lax.fori_loop(0, NGRP3, _group, 0)

        @pl.when(wid <= 1)
        def _tail(table=table, idx_h=idx_h, out_h=out_h):
            base = (NB3 * NGRP3 * 32 + wid) * C3
            pltpu.sync_copy(idx_h.at[pl.ds(base, C3)], idxs[0])
            pltpu.async_copy(table.at[idxs[0]], bufs[0], sem_g).wait()
            pltpu.sync_copy(bufs[0], out_h.at[pl.ds(base, C3)])


@functools.lru_cache(maxsize=1)
def _sc_kernels():
    mesh = plsc.VectorSubcoreMesh(core_axis_name="c", subcore_axis_name="s")

    def vmems(n, shape, dtype):
        return [pltpu.VMEM(shape, dtype) for _ in range(n)]

    def edge_scatter_wrap(es, row, col, z2, z1, ones1):
        body = lambda *args: _sc_edge_scatter_body(
            *args[:11],
            list(args[11:11 + NL]), list(args[11 + NL:11 + 2 * NL]),
            list(args[11 + 2 * NL:11 + 3 * NL]),
            *args[11 + 3 * NL:])
        return pl.kernel(
            body,
            mesh=mesh,
            out_type=(
                jax.ShapeDtypeStruct((N, HID), jnp.float32),
                jax.ShapeDtypeStruct((NPAD,), jnp.float32),
            ),
            scratch_types=(
                vmems(NL, (CHUNK, SLAB), jnp.float32)
                + vmems(NL, (CHUNK,), jnp.int32)
                + vmems(NL, (CHUNK,), jnp.int32)
                + [pltpu.VMEM((CHUNK,), jnp.float32),
                   pltpu.VMEM_SHARED((NPAD, SLAB), jnp.float32),
                   pltpu.VMEM_SHARED((NPAD,), jnp.float32),
                   pltpu.SemaphoreType.DMA, pltpu.SemaphoreType.DMA,
                   pltpu.SemaphoreType.DMA]
            ),
        )(*es, row, col, z2, z1, ones1)

    def gin_agg_wrap(h4, row, col, z2):
        body = lambda *args: _sc_gin_agg_body(
            *args[:5],
            list(args[5:5 + NL]), list(args[5 + NL:5 + 2 * NL]),
            list(args[5 + 2 * NL:5 + 3 * NL]),
            list(args[5 + 3 * NL:5 + 4 * NL]),
            *args[5 + 4 * NL:])
        return pl.kernel(
            body,
            mesh=mesh,
            out_type=jax.ShapeDtypeStruct((N, HID), jnp.float32),
            scratch_types=(
                vmems(NL, (CHUNK, SLAB), jnp.float32)
                + vmems(NL, (CHUNK,), jnp.int32)
                + vmems(NL, (CHUNK,), jnp.int32)
                + vmems(NL, (CHUNK,), jnp.int32)
                + [pltpu.VMEM_SHARED((NPAD, SLAB), jnp.float32),
                   pltpu.SemaphoreType.DMA, pltpu.SemaphoreType.DMA,
                   pltpu.SemaphoreType.DMA]
            ),
        )(h4, row, col, z2)

    def edge_gather_wrap(hs, hd, row, col):
        body = lambda *args: _sc_edge_gather_body(
            *args[:6],
            list(args[6:6 + NB3]), list(args[6 + NB3:6 + 2 * NB3]),
            *args[6 + 2 * NB3:])
        return pl.kernel(
            body,
            mesh=mesh,
            out_type=(
                jax.ShapeDtypeStruct((E2, HID), jnp.float32),
                jax.ShapeDtypeStruct((E2, HID), jnp.float32),
            ),
            scratch_types=(
                vmems(NB3, (C3, HID), jnp.float32)
                + vmems(NB3, (C3,), jnp.int32)
                + [pltpu.SemaphoreType.DMA, pltpu.SemaphoreType.DMA,
                   pltpu.SemaphoreType.DMA]
            ),
        )(hs, hd, row, col)

    return edge_scatter_wrap, gin_agg_wrap, edge_gather_wrap


def _dot(a, b):
    return jnp.dot(a, b, preferred_element_type=jnp.float32)


def _full(shape):
    return pl.BlockSpec(shape, lambda i: (0, 0))


def _mlp2_body(x_ref, w1_ref, b1_ref, w2_ref, b2_ref, o_ref):
    z = jnp.maximum(_dot(x_ref[...], w1_ref[...]) + b1_ref[...], 0.0)
    o_ref[...] = _dot(z, w2_ref[...]) + b2_ref[...]


def _mlp2(x, w1, b1, w2, b2, blk):
    m, k = x.shape
    return pl.pallas_call(
        _mlp2_body,
        grid=(m // blk,),
        in_specs=[
            pl.BlockSpec((blk, k), lambda i: (i, 0)),
            _full(w1.shape), _full(b1.shape),
            _full(w2.shape), _full(b2.shape),
        ],
        out_specs=pl.BlockSpec((blk, w2.shape[1]), lambda i: (i, 0)),
        out_shape=jax.ShapeDtypeStruct((m, w2.shape[1]), jnp.float32),
    )(x, w1, b1, w2, b2)


def _edge_enc_body(x_ref, w_ref, b_ref, *o_refs):
    z = jnp.maximum(_dot(x_ref[...], w_ref[...]) + b_ref[...], 0.0)
    for k, o_ref in enumerate(o_refs):
        o_ref[...] = z[:, k * SLAB:(k + 1) * SLAB]


def _edge_enc(x, w, b, blk):
    spec = pl.BlockSpec((blk, SLAB), lambda i: (i, 0))
    shp = jax.ShapeDtypeStruct((E, SLAB), jnp.float32)
    return pl.pallas_call(
        _edge_enc_body,
        grid=(E // blk,),
        in_specs=[
            pl.BlockSpec((blk, EF), lambda i: (i, 0)),
            _full(w.shape), _full(b.shape),
        ],
        out_specs=(spec,) * NSLAB,
        out_shape=(shp,) * NSLAB,
    )(x, w, b)


def _fusion_body(h_ref, agg_ref, cnt_ref, w1a_ref, w1b_ref, b1_ref,
                 w2_ref, b2_ref, o_ref):
    cnt = jnp.maximum(cnt_ref[...], 1.0)
    agg = agg_ref[...] / cnt
    z = jnp.maximum(_dot(h_ref[...], w1a_ref[...])
                    + _dot(agg, w1b_ref[...]) + b1_ref[...], 0.0)
    o_ref[...] = _dot(z, w2_ref[...]) + b2_ref[...]


def _fusion(h, agg, cnt, w1a, w1b, b1, w2, b2, blk):
    return pl.pallas_call(
        _fusion_body,
        grid=(N // blk,),
        in_specs=[
            pl.BlockSpec((blk, HID), lambda i: (i, 0)),
            pl.BlockSpec((blk, HID), lambda i: (i, 0)),
            pl.BlockSpec((blk, 1), lambda i: (i, 0)),
            _full(w1a.shape), _full(w1b.shape), _full(b1.shape),
            _full(w2.shape), _full(b2.shape),
        ],
        out_specs=pl.BlockSpec((blk, HID), lambda i: (i, 0)),
        out_shape=jax.ShapeDtypeStruct((N, HID), jnp.float32),
    )(h, agg, cnt, w1a, w1b, b1, w2, b2)


def _gin_body(h_ref, nbr_ref, eps_ref, w1_ref, b1_ref, w2_ref, b2_ref,
              a3_ref, c3_ref, o_ref):
    z = eps_ref[...] * h_ref[...] + nbr_ref[...]
    z = jnp.maximum(_dot(z, w1_ref[...]) + b1_ref[...], 0.0)
    z = jnp.maximum(_dot(z, w2_ref[...]) + b2_ref[...], 0.0)
    o_ref[...] = z * a3_ref[...] + c3_ref[...]


def _gin_mlp(h, nbr, epsp, w1, b1, w2, b2, a3, c3, blk):
    return pl.pallas_call(
        _gin_body,
        grid=(N // blk,),
        in_specs=[
            pl.BlockSpec((blk, HID), lambda i: (i, 0)),
            pl.BlockSpec((blk, HID), lambda i: (i, 0)),
            _full(epsp.shape),
            _full(w1.shape), _full(b1.shape),
            _full(w2.shape), _full(b2.shape),
            _full(a3.shape), _full(c3.shape),
        ],
        out_specs=pl.BlockSpec((blk, HID), lambda i: (i, 0)),
        out_shape=jax.ShapeDtypeStruct((N, HID), jnp.float32),
    )(h, nbr, epsp, w1, b1, w2, b2, a3, c3)


def _prep_body(h_ref, ws_ref, wd_ref, os_ref, od_ref):
    h = h_ref[...]
    os_ref[...] = _dot(h, ws_ref[...])
    od_ref[...] = _dot(h, wd_ref[...])


def _prep(h, ws, wd, blk):
    return pl.pallas_call(
        _prep_body,
        grid=(N // blk,),
        in_specs=[
            pl.BlockSpec((blk, HID), lambda i: (i, 0)),
            _full(ws.shape), _full(wd.shape),
        ],
        out_specs=(
            pl.BlockSpec((blk, HID), lambda i: (i, 0)),
            pl.BlockSpec((blk, HID), lambda i: (i, 0)),
        ),
        out_shape=(
            jax.ShapeDtypeStruct((N, HID), jnp.float32),
            jax.ShapeDtypeStruct((N, HID), jnp.float32),
        ),
    )(h, ws, wd)


def _cls_body(s_ref, d_ref, b1_ref, w2_ref, b2_ref, w3_ref, b3_ref,
              w4_ref, b4_ref, o_ref):
    z = jnp.maximum(s_ref[...] + d_ref[...] + b1_ref[...], 0.0)
    z = jnp.maximum(_dot(z, w2_ref[...]) + b2_ref[...], 0.0)
    z = jnp.maximum(_dot(z, w3_ref[...]) + b3_ref[...], 0.0)
    o_ref[...] = _dot(z, w4_ref[...]) + b4_ref[...]


def _cls(s, d, b1, w2, b2, w3, b3, w4, b4, blk):
    m = s.shape[0]
    return pl.pallas_call(
        _cls_body,
        grid=(m // blk,),
        in_specs=[
            pl.BlockSpec((blk, HID), lambda i: (i, 0)),
            pl.BlockSpec((blk, HID), lambda i: (i, 0)),
            _full(b1.shape),
            _full(w2.shape), _full(b2.shape),
            _full(w3.shape), _full(b3.shape),
            _full(w4.shape), _full(b4.shape),
        ],
        out_specs=pl.BlockSpec((blk, 2), lambda i: (i, 0)),
        out_shape=jax.ShapeDtypeStruct((m, 2), jnp.float32),
    )(s, d, b1, w2, b2, w3, b3, w4, b4)


def kernel(x, edge_index, edge_attr, params):
    p = params
    row = edge_index[0]
    col = edge_index[1]
    r2 = lambda v: v.reshape(1, -1)
    sc_edge_scatter, sc_gin_agg, sc_edge_gather = _sc_kernels()

    ne_s = p['ne_g1'] * _BNS
    h = _mlp2(x, p['ne_W1'] * ne_s[None, :],
              r2(p['ne_b1'] * ne_s + p['ne_be1']),
              p['ne_W2'], r2(p['ne_b2']), blk=1000)

    ee_s = p['ee_g'] * _BNS
    es = _edge_enc(edge_attr, p['ee_W'] * ee_s[None, :],
                   r2(p['ee_b'] * ee_s + p['ee_be']), blk=2000)

    z2 = jnp.zeros((RPT, SLAB), jnp.float32)
    z1 = jnp.zeros((RPT,), jnp.float32)
    ones1 = jnp.ones((CHUNK,), jnp.float32)
    agg, cnt_pad = sc_edge_scatter(es, row, col, z2, z1, ones1)
    cnt = cnt_pad[:N].reshape(N, 1)

    h = _fusion(h, agg, cnt,
                p['ef_W1'][:HID], p['ef_W1'][HID:], r2(p['ef_b1']),
                p['ef_W2'], r2(p['ef_b2']), blk=1000)

    for i in range(3):
        nbr = sc_gin_agg(h.reshape(NSLAB * N, SLAB), row, col, z2)
        s1 = p['g%d_g1' % i] * _BNS
        s2 = p['g%d_g2' % i] * _BNS
        h = _gin_mlp(
            h, nbr, (1.0 + p['eps%d' % i]).reshape(1, 1),
            p['g%d_W1' % i] * s1[None, :],
            r2(p['g%d_b1' % i] * s1 + p['g%d_be1' % i]),
            p['g%d_W2' % i] * s2[None, :],
            r2(p['g%d_b2' % i] * s2 + p['g%d_be2' % i]),
            r2(p['bn%d_g' % i] * _BNS), r2(p['bn%d_b' % i]), blk=1000)

    c1 = p['c_g1'] * _BNS
    wt, wb = p['c_W1'][:HID], p['c_W1'][HID:]
    hs, hd = _prep(h, (wt + wb) * c1[None, :], (wt - wb) * c1[None, :],
                   blk=1000)

    c2 = p['c_g2'] * _BNS
    cls_args = (
        r2(p['c_b1'] * c1 + p['c_be1']),
        p['c_W2'] * c2[None, :], r2(p['c_b2'] * c2 + p['c_be2']),
        p['c_W3'], r2(p['c_b3']),
        p['c_W4'], r2(p['c_b4']),
    )
    outs = []
    for lo in (0, E2):
        s_rows, d_rows = sc_edge_gather(hs, hd, row[lo:lo + E2],
                                        col[lo:lo + E2])
        outs.append(_cls(s_rows, d_rows, *cls_args, blk=2000))
    return jnp.concatenate(outs, axis=0)


# speedup vs baseline: 1.5144x; 1.0031x over previous
import functools

import jax
import jax.numpy as jnp
from jax import lax
from jax.experimental import pallas as pl
from jax.experimental.pallas import tpu as pltpu
from jax.experimental.pallas import tpu_sc as plsc

HID = 512
NF = 256
EF = 16
N = 10000
E = 160000

_BNS = (1.0 + 1e-5) ** -0.5

SLAB = 128
NSLAB = HID // SLAB
CHUNK = 80
NCHUNK = E // CHUNK
CPT = NCHUNK // 16
RPT = 640
NPAD = 16 * RPT
NL = 4
NGRP = 31

C3 = 64
E2 = E // 2
NC3 = E2 // C3
NB3 = 3
NGRP3 = 13


def _sc_edge_scatter_body(e0, e1, e2, e3, row_h, col_h, z2_h, z1_h, ones_h,
                          agg_h, cnt_h,
                          evs, ridxs, cidxs, onesv, slab_s, cnt_s,
                          sem_i, sem_e, sem_s):
    cid = lax.axis_index("c")
    sid = lax.axis_index("s")
    r0 = sid * RPT
    c0 = sid * CPT
    e_all = (e0, e1, e2, e3)
    pltpu.sync_copy(ones_h, onesv)

    for p in range(NSLAB):
        @pl.when(cid == p // 2)
        def _pass(p=p):
            e_p = e_all[p]
            pltpu.sync_copy(z2_h, slab_s.at[pl.ds(r0, RPT), :])
            if p == 0:
                pltpu.sync_copy(z1_h, cnt_s.at[pl.ds(r0, RPT)])
            plsc.subcore_barrier()

            def _group(g, carry):
                fires = []
                for b in range(NL):
                    @pl.when(g > 0)
                    def _drain(b=b):
                        pltpu.make_async_copy(
                            evs[b], slab_s.at[ridxs[b]], sem_s).wait()
                        pltpu.make_async_copy(
                            evs[b], slab_s.at[cidxs[b]], sem_s).wait()
                    base = (c0 + g * NL + b) * CHUNK
                    fires.append(pltpu.async_copy(
                        row_h.at[pl.ds(base, CHUNK)], ridxs[b], sem_i))
                    fires.append(pltpu.async_copy(
                        col_h.at[pl.ds(base, CHUNK)], cidxs[b], sem_i))
                    fires.append(pltpu.async_copy(
                        e_p.at[pl.ds(base, CHUNK)], evs[b], sem_e))
                for f in fires:
                    f.wait()
                for b in range(NL):
                    pltpu.async_copy(
                        evs[b], slab_s.at[ridxs[b]], sem_s, add=True)
                    pltpu.async_copy(
                        evs[b], slab_s.at[cidxs[b]], sem_s, add=True)
                return carry

            lax.fori_loop(0, NGRP, _group, 0)
            for b in range(NL):
                pltpu.make_async_copy(
                    evs[b], slab_s.at[ridxs[b]], sem_s).wait()
                pltpu.make_async_copy(
                    evs[b], slab_s.at[cidxs[b]], sem_s).wait()

            base = (c0 + NL * NGRP) * CHUNK
            pltpu.sync_copy(row_h.at[pl.ds(base, CHUNK)], ridxs[0])
            pltpu.sync_copy(col_h.at[pl.ds(base, CHUNK)], cidxs[0])
            pltpu.sync_copy(e_p.at[pl.ds(base, CHUNK)], evs[0])
            pltpu.sync_copy(evs[0], slab_s.at[ridxs[0]], add=True)
            pltpu.sync_copy(evs[0], slab_s.at[cidxs[0]], add=True)

            plsc.subcore_barrier()

            @pl.when(sid < 15)
            def _write_full():
                pltpu.sync_copy(
                    slab_s.at[pl.ds(r0, RPT), :],
                    agg_h.at[pl.ds(r0, RPT), pl.ds(p * SLAB, SLAB)])

            @pl.when(sid == 15)
            def _write_tail():
                nr = N - 15 * RPT
                pltpu.sync_copy(
                    slab_s.at[pl.ds(15 * RPT, nr), :],
                    agg_h.at[pl.ds(15 * RPT, nr), pl.ds(p * SLAB, SLAB)])

            plsc.subcore_barrier()

    @pl.when(cid == 0)
    def _cnt():
        def _group(g, carry):
            fires = []
            for b in range(NL):
                base = (c0 + g * NL + b) * CHUNK
                fires.append(pltpu.async_copy(
                    row_h.at[pl.ds(base, CHUNK)], ridxs[b], sem_i))
                fires.append(pltpu.async_copy(
                    col_h.at[pl.ds(base, CHUNK)], cidxs[b], sem_i))
            for f in fires:
                f.wait()
            scat = []
            for b in range(NL):
                scat.append(pltpu.async_copy(
                    onesv, cnt_s.at[ridxs[b]], sem_s, add=True))
                scat.append(pltpu.async_copy(
                    onesv, cnt_s.at[cidxs[b]], sem_s, add=True))
            for f in scat:
                f.wait()
            return carry

        lax.fori_loop(0, NGRP, _group, 0)
        base = (c0 + NL * NGRP) * CHUNK
        pltpu.sync_copy(row_h.at[pl.ds(base, CHUNK)], ridxs[0])
        pltpu.sync_copy(col_h.at[pl.ds(base, CHUNK)], cidxs[0])
        pltpu.sync_copy(onesv, cnt_s.at[ridxs[0]], add=True)
        pltpu.sync_copy(onesv, cnt_s.at[cidxs[0]], add=True)
        plsc.subcore_barrier()
        pltpu.sync_copy(cnt_s.at[pl.ds(r0, RPT)], cnt_h.at[pl.ds(r0, RPT)])


def _sc_gin_agg_body(h4_h, row_h, col_h, z2_h,
                     nbr_h,
                     evs, rjoint, cidxs, gidxs, slab_s,
                     sem_i, sem_g, sem_s):
    cid = lax.axis_index("c")
    sid = lax.axis_index("s")
    r0 = sid * RPT
    c0 = sid * CPT

    for p in range(NSLAB):
        @pl.when(cid == p // 2)
        def _pass(p=p):
            pltpu.sync_copy(z2_h, slab_s.at[pl.ds(r0, RPT), :])
            plsc.subcore_barrier()

            def _group(g, carry):
                gbase = (c0 + g * NL) * CHUNK
                ifires = [pltpu.async_copy(
                    row_h.at[pl.ds(gbase, NL * CHUNK)], rjoint, sem_i)]
                for b in range(NL):
                    @pl.when(g > 0)
                    def _drain(b=b):
                        pltpu.make_async_copy(
                            evs[b], slab_s.at[cidxs[b]], sem_s).wait()
                    base = (c0 + g * NL + b) * CHUNK
                    ifires.append(pltpu.async_copy(
                        col_h.at[pl.ds(base, CHUNK)], cidxs[b], sem_i))
                for f in ifires:
                    f.wait()
                gath = []
                for b in range(NL):
                    for k in range(CHUNK // 16):
                        gidxs[b][pl.ds(k * 16, 16)] = (
                            rjoint[pl.ds(b * CHUNK + k * 16, 16)] * 4 + p)
                    gath.append(pltpu.async_copy(
                        h4_h.at[gidxs[b]], evs[b], sem_g))
                for b in range(NL):
                    gath[b].wait()
                    pltpu.async_copy(
                        evs[b], slab_s.at[cidxs[b]], sem_s, add=True)
                return carry

            lax.fori_loop(0, NGRP, _group, 0)
            for b in range(NL):
                pltpu.make_async_copy(
                    evs[b], slab_s.at[cidxs[b]], sem_s).wait()

            base = (c0 + NL * NGRP) * CHUNK
            pltpu.sync_copy(row_h.at[pl.ds(base, CHUNK)],
                            rjoint.at[pl.ds(0, CHUNK)])
            pltpu.sync_copy(col_h.at[pl.ds(base, CHUNK)], cidxs[0])
            for k in range(CHUNK // 16):
                gidxs[0][pl.ds(k * 16, 16)] = (
                    rjoint[pl.ds(k * 16, 16)] * 4 + p)
            pltpu.async_copy(h4_h.at[gidxs[0]], evs[0], sem_g).wait()
            pltpu.sync_copy(evs[0], slab_s.at[cidxs[0]], add=True)

            plsc.subcore_barrier()

            @pl.when(sid < 15)
            def _write_full():
                pltpu.sync_copy(
                    slab_s.at[pl.ds(r0, RPT), :],
                    nbr_h.at[pl.ds(r0, RPT), pl.ds(p * SLAB, SLAB)])

            @pl.when(sid == 15)
            def _write_tail():
                nr = N - 15 * RPT
                pltpu.sync_copy(
                    slab_s.at[pl.ds(15 * RPT, nr), :],
                    nbr_h.at[pl.ds(15 * RPT, nr), pl.ds(p * SLAB, SLAB)])

            plsc.subcore_barrier()


def _sc_edge_gather_body(hs_h, hd_h, row_h, col_h,
                         s_out_h, d_out_h,
                         bufs, idxs,
                         sem_i, sem_g, sem_w):
    cid = lax.axis_index("c")
    sid = lax.axis_index("s")
    wid = sid * 2 + cid

    for table, idx_h, out_h in ((hs_h, row_h, s_out_h),
                                (hd_h, col_h, d_out_h)):
        def _group(g, carry, table=table, idx_h=idx_h, out_h=out_h):
            fires = []
            for b in range(NB3):
                base = (wid + (g * NB3 + b) * 32) * C3
                fires.append(pltpu.async_copy(
                    idx_h.at[pl.ds(base, C3)], idxs[b], sem_i))
            for f in fires:
                f.wait()
            gath = [pltpu.async_copy(table.at[idxs[b]], bufs[b], sem_g)
                    for b in range(NB3)]
            for f in gath:
                f.wait()
            wr = []
            for b in range(NB3):
                base = (wid + (g * NB3 + b) * 32) * C3
                wr.append(pltpu.async_copy(
                    bufs[b], out_h.at[pl.ds(base, C3)], sem_w))
            for f in wr:
                f.wait()
            return carry

        lax.fori_loop(0, NGRP3, _group, 0)

        @pl.when(wid <= 1)
        def _tail(table=table, idx_h=idx_h, out_h=out_h):
            base = (NB3 * NGRP3 * 32 + wid) * C3
            pltpu.sync_copy(idx_h.at[pl.ds(base, C3)], idxs[0])
            pltpu.async_copy(table.at[idxs[0]], bufs[0], sem_g).wait()
            pltpu.sync_copy(bufs[0], out_h.at[pl.ds(base, C3)])


@functools.lru_cache(maxsize=1)
def _sc_kernels():
    mesh = plsc.VectorSubcoreMesh(core_axis_name="c", subcore_axis_name="s")

    def vmems(n, shape, dtype):
        return [pltpu.VMEM(shape, dtype) for _ in range(n)]

    def edge_scatter_wrap(es, row, col, z2, z1, ones1):
        body = lambda *args: _sc_edge_scatter_body(
            *args[:11],
            list(args[11:11 + NL]), list(args[11 + NL:11 + 2 * NL]),
            list(args[11 + 2 * NL:11 + 3 * NL]),
            *args[11 + 3 * NL:])
        return pl.kernel(
            body,
            mesh=mesh,
            out_type=(
                jax.ShapeDtypeStruct((N, HID), jnp.float32),
                jax.ShapeDtypeStruct((NPAD,), jnp.float32),
            ),
            scratch_types=(
                vmems(NL, (CHUNK, SLAB), jnp.float32)
                + vmems(NL, (CHUNK,), jnp.int32)
                + vmems(NL, (CHUNK,), jnp.int32)
                + [pltpu.VMEM((CHUNK,), jnp.float32),
                   pltpu.VMEM_SHARED((NPAD, SLAB), jnp.float32),
                   pltpu.VMEM_SHARED((NPAD,), jnp.float32),
                   pltpu.SemaphoreType.DMA, pltpu.SemaphoreType.DMA,
                   pltpu.SemaphoreType.DMA]
            ),
        )(*es, row, col, z2, z1, ones1)

    def gin_agg_wrap(h4, row, col, z2):
        body = lambda *args: _sc_gin_agg_body(
            *args[:5],
            list(args[5:5 + NL]), args[5 + NL],
            list(args[6 + NL:6 + 2 * NL]),
            list(args[6 + 2 * NL:6 + 3 * NL]),
            *args[6 + 3 * NL:])
        return pl.kernel(
            body,
            mesh=mesh,
            out_type=jax.ShapeDtypeStruct((N, HID), jnp.float32),
            scratch_types=(
                vmems(NL, (CHUNK, SLAB), jnp.float32)
                + [pltpu.VMEM((NL * CHUNK,), jnp.int32)]
                + vmems(NL, (CHUNK,), jnp.int32)
                + vmems(NL, (CHUNK,), jnp.int32)
                + [pltpu.VMEM_SHARED((NPAD, SLAB), jnp.float32),
                   pltpu.SemaphoreType.DMA, pltpu.SemaphoreType.DMA,
                   pltpu.SemaphoreType.DMA]
            ),
        )(h4, row, col, z2)

    def edge_gather_wrap(hs, hd, row, col):
        body = lambda *args: _sc_edge_gather_body(
            *args[:6],
            list(args[6:6 + NB3]), list(args[6 + NB3:6 + 2 * NB3]),
            *args[6 + 2 * NB3:])
        return pl.kernel(
            body,
            mesh=mesh,
            out_type=(
                jax.ShapeDtypeStruct((E2, HID), jnp.float32),
                jax.ShapeDtypeStruct((E2, HID), jnp.float32),
            ),
            scratch_types=(
                vmems(NB3, (C3, HID), jnp.float32)
                + vmems(NB3, (C3,), jnp.int32)
                + [pltpu.SemaphoreType.DMA, pltpu.SemaphoreType.DMA,
                   pltpu.SemaphoreType.DMA]
            ),
        )(hs, hd, row, col)

    return edge_scatter_wrap, gin_agg_wrap, edge_gather_wrap


def _dot(a, b):
    return jnp.dot(a, b, preferred_element_type=jnp.float32)


def _full(shape):
    return pl.BlockSpec(shape, lambda i: (0, 0))


def _mlp2_body(x_ref, w1_ref, b1_ref, w2_ref, b2_ref, o_ref):
    z = jnp.maximum(_dot(x_ref[...], w1_ref[...]) + b1_ref[...], 0.0)
    o_ref[...] = _dot(z, w2_ref[...]) + b2_ref[...]


def _mlp2(x, w1, b1, w2, b2, blk):
    m, k = x.shape
    return pl.pallas_call(
        _mlp2_body,
        grid=(m // blk,),
        in_specs=[
            pl.BlockSpec((blk, k), lambda i: (i, 0)),
            _full(w1.shape), _full(b1.shape),
            _full(w2.shape), _full(b2.shape),
        ],
        out_specs=pl.BlockSpec((blk, w2.shape[1]), lambda i: (i, 0)),
        out_shape=jax.ShapeDtypeStruct((m, w2.shape[1]), jnp.float32),
    )(x, w1, b1, w2, b2)


def _edge_enc_body(x_ref, w_ref, b_ref, *o_refs):
    z = jnp.maximum(_dot(x_ref[...], w_ref[...]) + b_ref[...], 0.0)
    for k, o_ref in enumerate(o_refs):
        o_ref[...] = z[:, k * SLAB:(k + 1) * SLAB]


def _edge_enc(x, w, b, blk):
    spec = pl.BlockSpec((blk, SLAB), lambda i: (i, 0))
    shp = jax.ShapeDtypeStruct((E, SLAB), jnp.float32)
    return pl.pallas_call(
        _edge_enc_body,
        grid=(E // blk,),
        in_specs=[
            pl.BlockSpec((blk, EF), lambda i: (i, 0)),
            _full(w.shape), _full(b.shape),
        ],
        out_specs=(spec,) * NSLAB,
        out_shape=(shp,) * NSLAB,
    )(x, w, b)


def _fusion_body(h_ref, agg_ref, cnt_ref, w1a_ref, w1b_ref, b1_ref,
                 w2_ref, b2_ref, o_ref):
    cnt = jnp.maximum(cnt_ref[...], 1.0)
    agg = agg_ref[...] / cnt
    z = jnp.maximum(_dot(h_ref[...], w1a_ref[...])
                    + _dot(agg, w1b_ref[...]) + b1_ref[...], 0.0)
    o_ref[...] = _dot(z, w2_ref[...]) + b2_ref[...]


def _fusion(h, agg, cnt, w1a, w1b, b1, w2, b2, blk):
    return pl.pallas_call(
        _fusion_body,
        grid=(N // blk,),
        in_specs=[
            pl.BlockSpec((blk, HID), lambda i: (i, 0)),
            pl.BlockSpec((blk, HID), lambda i: (i, 0)),
            pl.BlockSpec((blk, 1), lambda i: (i, 0)),
            _full(w1a.shape), _full(w1b.shape), _full(b1.shape),
            _full(w2.shape), _full(b2.shape),
        ],
        out_specs=pl.BlockSpec((blk, HID), lambda i: (i, 0)),
        out_shape=jax.ShapeDtypeStruct((N, HID), jnp.float32),
    )(h, agg, cnt, w1a, w1b, b1, w2, b2)


def _gin_body(h_ref, nbr_ref, eps_ref, w1_ref, b1_ref, w2_ref, b2_ref,
              a3_ref, c3_ref, o_ref):
    z = eps_ref[...] * h_ref[...] + nbr_ref[...]
    z = jnp.maximum(_dot(z, w1_ref[...]) + b1_ref[...], 0.0)
    z = jnp.maximum(_dot(z, w2_ref[...]) + b2_ref[...], 0.0)
    o_ref[...] = z * a3_ref[...] + c3_ref[...]


def _gin_mlp(h, nbr, epsp, w1, b1, w2, b2, a3, c3, blk):
    return pl.pallas_call(
        _gin_body,
        grid=(N // blk,),
        in_specs=[
            pl.BlockSpec((blk, HID), lambda i: (i, 0)),
            pl.BlockSpec((blk, HID), lambda i: (i, 0)),
            _full(epsp.shape),
            _full(w1.shape), _full(b1.shape),
            _full(w2.shape), _full(b2.shape),
            _full(a3.shape), _full(c3.shape),
        ],
        out_specs=pl.BlockSpec((blk, HID), lambda i: (i, 0)),
        out_shape=jax.ShapeDtypeStruct((N, HID), jnp.float32),
    )(h, nbr, epsp, w1, b1, w2, b2, a3, c3)


def _prep_body(h_ref, ws_ref, wd_ref, os_ref, od_ref):
    h = h_ref[...]
    os_ref[...] = _dot(h, ws_ref[...])
    od_ref[...] = _dot(h, wd_ref[...])


def _prep(h, ws, wd, blk):
    return pl.pallas_call(
        _prep_body,
        grid=(N // blk,),
        in_specs=[
            pl.BlockSpec((blk, HID), lambda i: (i, 0)),
            _full(ws.shape), _full(wd.shape),
        ],
        out_specs=(
            pl.BlockSpec((blk, HID), lambda i: (i, 0)),
            pl.BlockSpec((blk, HID), lambda i: (i, 0)),
        ),
        out_shape=(
            jax.ShapeDtypeStruct((N, HID), jnp.float32),
            jax.ShapeDtypeStruct((N, HID), jnp.float32),
        ),
    )(h, ws, wd)


def _cls_body(s_ref, d_ref, b1_ref, w2_ref, b2_ref, w3_ref, b3_ref,
              w4_ref, b4_ref, o_ref):
    z = jnp.maximum(s_ref[...] + d_ref[...] + b1_ref[...], 0.0)
    z = jnp.maximum(_dot(z, w2_ref[...]) + b2_ref[...], 0.0)
    z = jnp.maximum(_dot(z, w3_ref[...]) + b3_ref[...], 0.0)
    o_ref[...] = _dot(z, w4_ref[...]) + b4_ref[...]


def _cls(s, d, b1, w2, b2, w3, b3, w4, b4, blk):
    m = s.shape[0]
    return pl.pallas_call(
        _cls_body,
        grid=(m // blk,),
        in_specs=[
            pl.BlockSpec((blk, HID), lambda i: (i, 0)),
            pl.BlockSpec((blk, HID), lambda i: (i, 0)),
            _full(b1.shape),
            _full(w2.shape), _full(b2.shape),
            _full(w3.shape), _full(b3.shape),
            _full(w4.shape), _full(b4.shape),
        ],
        out_specs=pl.BlockSpec((blk, 2), lambda i: (i, 0)),
        out_shape=jax.ShapeDtypeStruct((m, 2), jnp.float32),
    )(s, d, b1, w2, b2, w3, b3, w4, b4)


def kernel(x, edge_index, edge_attr, params):
    p = params
    row = edge_index[0]
    col = edge_index[1]
    r2 = lambda v: v.reshape(1, -1)
    sc_edge_scatter, sc_gin_agg, sc_edge_gather = _sc_kernels()

    ne_s = p['ne_g1'] * _BNS
    h = _mlp2(x, p['ne_W1'] * ne_s[None, :],
              r2(p['ne_b1'] * ne_s + p['ne_be1']),
              p['ne_W2'], r2(p['ne_b2']), blk=1000)

    ee_s = p['ee_g'] * _BNS
    es = _edge_enc(edge_attr, p['ee_W'] * ee_s[None, :],
                   r2(p['ee_b'] * ee_s + p['ee_be']), blk=2000)

    z2 = jnp.zeros((RPT, SLAB), jnp.float32)
    z1 = jnp.zeros((RPT,), jnp.float32)
    ones1 = jnp.ones((CHUNK,), jnp.float32)
    agg, cnt_pad = sc_edge_scatter(es, row, col, z2, z1, ones1)
    cnt = cnt_pad[:N].reshape(N, 1)

    h = _fusion(h, agg, cnt,
                p['ef_W1'][:HID], p['ef_W1'][HID:], r2(p['ef_b1']),
                p['ef_W2'], r2(p['ef_b2']), blk=1000)

    for i in range(3):
        nbr = sc_gin_agg(h.reshape(NSLAB * N, SLAB), row, col, z2)
        s1 = p['g%d_g1' % i] * _BNS
        s2 = p['g%d_g2' % i] * _BNS
        h = _gin_mlp(
            h, nbr, (1.0 + p['eps%d' % i]).reshape(1, 1),
            p['g%d_W1' % i] * s1[None, :],
            r2(p['g%d_b1' % i] * s1 + p['g%d_be1' % i]),
            p['g%d_W2' % i] * s2[None, :],
            r2(p['g%d_b2' % i] * s2 + p['g%d_be2' % i]),
            r2(p['bn%d_g' % i] * _BNS), r2(p['bn%d_b' % i]), blk=1000)

    c1 = p['c_g1'] * _BNS
    wt, wb = p['c_W1'][:HID], p['c_W1'][HID:]
    hs, hd = _prep(h, (wt + wb) * c1[None, :], (wt - wb) * c1[None, :],
                   blk=1000)

    c2 = p['c_g2'] * _BNS
    cls_args = (
        r2(p['c_b1'] * c1 + p['c_be1']),
        p['c_W2'] * c2[None, :], r2(p['c_b2'] * c2 + p['c_be2']),
        p['c_W3'], r2(p['c_b3']),
        p['c_W4'], r2(p['c_b4']),
    )
    outs = []
    for lo in (0, E2):
        s_rows, d_rows = sc_edge_gather(hs, hd, row[lo:lo + E2],
                                        col[lo:lo + E2])
        outs.append(_cls(s_rows, d_rows, *cls_args, blk=2000))
    return jnp.concatenate(outs, axis=0)
